# baseline (speedup 1.0000x reference)
import functools

import jax
import jax.numpy as jnp
import numpy as np
from jax import lax
from jax.experimental import pallas as pl
from jax.experimental.pallas import tpu as pltpu
from jax.experimental.pallas import tpu_sc as plsc

N_EMB = 1000000
D_EMB = 64
BATCH = 16384

NC = 2
NS = 16
L = 16
NW = NC * NS
B_PER_W = BATCH // NW
N_CHUNK = B_PER_W // 128

SC_BLK = 768
SC_NB = 16
SC_COLS = SC_BLK * SC_NB
SPLIT = SC_COLS * NW
BLK = 49152
TC_OFF = SPLIT // BLK
N_BLK = 13
ZA_LEN = N_BLK * BLK


def _mv_body(p_ref, w_ref, b_ref, z_ref):
    z = lax.dot_general(w_ref[...], p_ref[...], (((1,), (0,)), ((), ())),
                        preferred_element_type=jnp.float32)
    z_ref[...] = z.reshape(BLK) + b_ref[0, 0]


@jax.jit
def _tc_matvec(p, w, b):
    return pl.pallas_call(
        _mv_body,
        grid=(N_BLK,),
        in_specs=[
            pl.BlockSpec((D_EMB, BLK), lambda c: (0, c + TC_OFF)),
            pl.BlockSpec((1, D_EMB), lambda c: (0, 0)),
            pl.BlockSpec((1, 1), lambda c: (0, 0)),
        ],
        out_specs=pl.BlockSpec((BLK,), lambda c: (c,)),
        out_shape=jax.ShapeDtypeStruct((ZA_LEN,), jnp.float32),
    )(p, w, b)


def _sc_mv_compute(j, buf, w_sc, b_vec, zb_v):
    def col16(q, carry):
        acc = b_vec
        for d in range(D_EMB):
            acc = acc + buf[d, pl.ds(q * L, L)] * w_sc[d]
        zb_v[pl.ds(j * SC_BLK + q * L, L)] = acc
        return carry

    lax.fori_loop(0, SC_BLK // L, col16, 0)


def _sc_mv_body(p_hbm, w_hbm, b_hbm, zb_hbm,
                w_v, b_v, buf0, buf1, zb_v, sem0, sem1):
    cid = lax.axis_index("c")
    sid = lax.axis_index("s")
    wid = sid * NC + cid
    tstart = wid * SC_COLS

    pltpu.sync_copy(w_hbm, w_v)
    pltpu.sync_copy(b_hbm, b_v)
    w_chunks = [w_v[pl.ds(c * L, L)] for c in range(D_EMB // L)]
    w_sc = [w_chunks[d // L][d % L] for d in range(D_EMB)]
    b_vec = b_v[...]

    def start(j, buf, sem):
        return pltpu.async_copy(
            p_hbm.at[:, pl.ds(tstart + j * SC_BLK, SC_BLK)], buf, sem)

    start(0, buf0, sem0)

    def group(i2, carry):
        j0 = i2 * 2
        c1 = start(j0 + 1, buf1, sem1)
        pltpu.make_async_copy(
            p_hbm.at[:, pl.ds(tstart + j0 * SC_BLK, SC_BLK)], buf0,
            sem0).wait()
        _sc_mv_compute(j0, buf0, w_sc, b_vec, zb_v)

        @pl.when(j0 + 2 < SC_NB)
        def _():
            start(j0 + 2, buf0, sem0)

        c1.wait()
        _sc_mv_compute(j0 + 1, buf1, w_sc, b_vec, zb_v)
        return carry

    lax.fori_loop(0, SC_NB // 2, group, 0)
    pltpu.sync_copy(zb_v, zb_hbm.at[pl.ds(tstart, SC_COLS)])


@jax.jit
def _sc_matvec(p, w_flat, b_vec):
    mesh = plsc.VectorSubcoreMesh(core_axis_name="c", subcore_axis_name="s")
    k = pl.kernel(
        _sc_mv_body,
        mesh=mesh,
        out_type=jax.ShapeDtypeStruct((SPLIT,), jnp.float32),
        scratch_types=[
            pltpu.VMEM((D_EMB,), jnp.float32),
            pltpu.VMEM((L,), jnp.float32),
            pltpu.VMEM((D_EMB, SC_BLK), jnp.float32),
            pltpu.VMEM((D_EMB, SC_BLK), jnp.float32),
            pltpu.VMEM((SC_COLS,), jnp.float32),
            pltpu.SemaphoreType.DMA,
            pltpu.SemaphoreType.DMA,
        ],
    )
    return k(p, w_flat, b_vec)


def _sc_g_body(idx_hbm, za_hbm, zb_hbm, out_hbm,
               idx_v, ia_v, ib_v, outa_v, outb_v, out_v, sem):
    cid = lax.axis_index("c")
    sid = lax.axis_index("s")
    wid = sid * NC + cid
    base = wid * B_PER_W

    pltpu.sync_copy(idx_hbm.at[wid], idx_v)
    for jj in range(N_CHUNK):
        for q in range(128 // L):
            ivec = idx_v[jj, pl.ds(q * L, L)]
            ia_v[jj, pl.ds(q * L, L)] = jnp.maximum(ivec - SPLIT, 0)
            ib_v[jj, pl.ds(q * L, L)] = jnp.minimum(ivec, SPLIT - 1)

    copies = []
    for jj in range(N_CHUNK):
        copies.append(pltpu.async_copy(
            za_hbm.at[ia_v.at[jj]], outa_v.at[pl.ds(jj * 128, 128)], sem))
        copies.append(pltpu.async_copy(
            zb_hbm.at[ib_v.at[jj]], outb_v.at[pl.ds(jj * 128, 128)], sem))
    for c in copies:
        c.wait()

    for jj in range(N_CHUNK):
        for q in range(128 // L):
            off = jj * 128 + q * L
            ivec = idx_v[jj, pl.ds(q * L, L)]
            av = outa_v[pl.ds(off, L)]
            bv = outb_v[pl.ds(off, L)]
            out_v[pl.ds(off, L)] = jnp.where(ivec < SPLIT, bv, av)

    pltpu.sync_copy(out_v, out_hbm.at[pl.ds(base, B_PER_W)])


@jax.jit
def _sc_gather(idx_r, za, zb):
    mesh = plsc.VectorSubcoreMesh(core_axis_name="c", subcore_axis_name="s")
    k = pl.kernel(
        _sc_g_body,
        mesh=mesh,
        compiler_params=pltpu.CompilerParams(use_tc_tiling_on_sc=False),
        out_type=jax.ShapeDtypeStruct((BATCH,), jnp.float32),
        scratch_types=[
            pltpu.VMEM((N_CHUNK, 128), jnp.int32),
            pltpu.VMEM((N_CHUNK, 128), jnp.int32),
            pltpu.VMEM((N_CHUNK, 128), jnp.int32),
            pltpu.VMEM((B_PER_W,), jnp.float32),
            pltpu.VMEM((B_PER_W,), jnp.float32),
            pltpu.VMEM((B_PER_W,), jnp.float32),
            pltpu.SemaphoreType.DMA,
        ],
    )
    return k(idx_r, za, zb)


def kernel(idx, table, W, b):
    p = table.T
    w = W.reshape(1, D_EMB).astype(jnp.float32)
    w_flat = W.reshape(D_EMB).astype(jnp.float32)
    b2 = b.reshape(1, 1).astype(jnp.float32)
    b_vec = jnp.broadcast_to(b.astype(jnp.float32), (L,))
    za = _tc_matvec(p, w, b2)
    zb = _sc_matvec(p, w_flat, b_vec)
    idx_r = idx.astype(jnp.int32).reshape(NW, N_CHUNK, 128)
    out = _sc_gather(idx_r, za, zb)
    return out.reshape(BATCH, 1)


# speedup vs baseline: 1.5164x; 1.5164x over previous
import functools

import jax
import jax.numpy as jnp
import numpy as np
from jax import lax
from jax.experimental import pallas as pl
from jax.experimental.pallas import tpu as pltpu
from jax.experimental.pallas import tpu_sc as plsc

N_EMB = 1000000
D_EMB = 64
BATCH = 16384

NC = 2
NS = 16
L = 16
NW = NC * NS
B_PER_W = BATCH // NW
N_CHUNK = B_PER_W // 128

SC_BLK = 768
SC_NB = 16
SC_COLS = SC_BLK * SC_NB
SPLIT = SC_COLS * NW
BLK = 49152
TC_OFF = SPLIT // BLK
N_BLK = 13
ZA_LEN = N_BLK * BLK


def _mv_body(p_ref, w_ref, b_ref, z_ref):
    z = lax.dot_general(w_ref[...], p_ref[...], (((1,), (0,)), ((), ())),
                        preferred_element_type=jnp.float32)
    z_ref[...] = z.reshape(BLK) + b_ref[0, 0]


@jax.jit
def _tc_matvec(p, w, b):
    return pl.pallas_call(
        _mv_body,
        grid=(N_BLK,),
        in_specs=[
            pl.BlockSpec((D_EMB, BLK), lambda c: (0, c + TC_OFF)),
            pl.BlockSpec((1, D_EMB), lambda c: (0, 0)),
            pl.BlockSpec((1, 1), lambda c: (0, 0)),
        ],
        out_specs=pl.BlockSpec((BLK,), lambda c: (c,)),
        out_shape=jax.ShapeDtypeStruct((ZA_LEN,), jnp.float32),
    )(p, w, b)


def _sc_mv_compute(j, buf, w_sc, b_vec, zb_v):
    def col16(q, carry):
        accs = [buf[k, pl.ds(q * L, L)] * w_sc[k] for k in range(4)]
        for d in range(4, D_EMB):
            k = d % 4
            accs[k] = accs[k] + buf[d, pl.ds(q * L, L)] * w_sc[d]
        acc = (accs[0] + accs[1]) + (accs[2] + accs[3]) + b_vec
        zb_v[pl.ds(j * SC_BLK + q * L, L)] = acc
        return carry

    lax.fori_loop(0, SC_BLK // L, col16, 0, unroll=2)


def _sc_mv_body(p_hbm, w_hbm, b_hbm, zb_hbm,
                w_v, b_v, buf0, buf1, zb_v, sem0, sem1):
    cid = lax.axis_index("c")
    sid = lax.axis_index("s")
    wid = sid * NC + cid
    tstart = wid * SC_COLS

    pltpu.sync_copy(w_hbm, w_v)
    pltpu.sync_copy(b_hbm, b_v)
    w_chunks = [w_v[pl.ds(c * L, L)] for c in range(D_EMB // L)]
    w_sc = [w_chunks[d // L][d % L] for d in range(D_EMB)]
    b_vec = b_v[...]

    def start(j, buf, sem):
        return pltpu.async_copy(
            p_hbm.at[:, pl.ds(tstart + j * SC_BLK, SC_BLK)], buf, sem)

    start(0, buf0, sem0)

    def group(i2, carry):
        j0 = i2 * 2
        c1 = start(j0 + 1, buf1, sem1)
        pltpu.make_async_copy(
            p_hbm.at[:, pl.ds(tstart + j0 * SC_BLK, SC_BLK)], buf0,
            sem0).wait()
        _sc_mv_compute(j0, buf0, w_sc, b_vec, zb_v)

        @pl.when(j0 + 2 < SC_NB)
        def _():
            start(j0 + 2, buf0, sem0)

        c1.wait()
        _sc_mv_compute(j0 + 1, buf1, w_sc, b_vec, zb_v)
        return carry

    lax.fori_loop(0, SC_NB // 2, group, 0)
    pltpu.sync_copy(zb_v, zb_hbm.at[pl.ds(tstart, SC_COLS)])


@jax.jit
def _sc_matvec(p, w_flat, b_vec):
    mesh = plsc.VectorSubcoreMesh(core_axis_name="c", subcore_axis_name="s")
    k = pl.kernel(
        _sc_mv_body,
        mesh=mesh,
        out_type=jax.ShapeDtypeStruct((SPLIT,), jnp.float32),
        scratch_types=[
            pltpu.VMEM((D_EMB,), jnp.float32),
            pltpu.VMEM((L,), jnp.float32),
            pltpu.VMEM((D_EMB, SC_BLK), jnp.float32),
            pltpu.VMEM((D_EMB, SC_BLK), jnp.float32),
            pltpu.VMEM((SC_COLS,), jnp.float32),
            pltpu.SemaphoreType.DMA,
            pltpu.SemaphoreType.DMA,
        ],
    )
    return k(p, w_flat, b_vec)


def _sc_g_body(idx_hbm, za_hbm, zb_hbm, out_hbm,
               idx_v, ia_v, ib_v, outa_v, outb_v, out_v, sem):
    cid = lax.axis_index("c")
    sid = lax.axis_index("s")
    wid = sid * NC + cid
    base = wid * B_PER_W

    pltpu.sync_copy(idx_hbm.at[wid], idx_v)
    lanes = lax.iota(jnp.int32, L)
    for jj in range(N_CHUNK):
        for q in range(128 // L):
            spread = base + jj * 128 + q * L + lanes
            ivec = idx_v[jj, pl.ds(q * L, L)]
            in_a = ivec >= SPLIT
            ia_v[jj, pl.ds(q * L, L)] = jnp.where(in_a, ivec - SPLIT, spread)
            ib_v[jj, pl.ds(q * L, L)] = jnp.where(in_a, spread, ivec)

    copies = []
    for jj in range(N_CHUNK):
        copies.append(pltpu.async_copy(
            za_hbm.at[ia_v.at[jj]], outa_v.at[pl.ds(jj * 128, 128)], sem))
        copies.append(pltpu.async_copy(
            zb_hbm.at[ib_v.at[jj]], outb_v.at[pl.ds(jj * 128, 128)], sem))
    for c in copies:
        c.wait()

    for jj in range(N_CHUNK):
        for q in range(128 // L):
            off = jj * 128 + q * L
            ivec = idx_v[jj, pl.ds(q * L, L)]
            av = outa_v[pl.ds(off, L)]
            bv = outb_v[pl.ds(off, L)]
            out_v[pl.ds(off, L)] = jnp.where(ivec < SPLIT, bv, av)

    pltpu.sync_copy(out_v, out_hbm.at[pl.ds(base, B_PER_W)])


@jax.jit
def _sc_gather(idx_r, za, zb):
    mesh = plsc.VectorSubcoreMesh(core_axis_name="c", subcore_axis_name="s")
    k = pl.kernel(
        _sc_g_body,
        mesh=mesh,
        compiler_params=pltpu.CompilerParams(use_tc_tiling_on_sc=False),
        out_type=jax.ShapeDtypeStruct((BATCH,), jnp.float32),
        scratch_types=[
            pltpu.VMEM((N_CHUNK, 128), jnp.int32),
            pltpu.VMEM((N_CHUNK, 128), jnp.int32),
            pltpu.VMEM((N_CHUNK, 128), jnp.int32),
            pltpu.VMEM((B_PER_W,), jnp.float32),
            pltpu.VMEM((B_PER_W,), jnp.float32),
            pltpu.VMEM((B_PER_W,), jnp.float32),
            pltpu.SemaphoreType.DMA,
        ],
    )
    return k(idx_r, za, zb)


def kernel(idx, table, W, b):
    p = table.T
    w = W.reshape(1, D_EMB).astype(jnp.float32)
    w_flat = W.reshape(D_EMB).astype(jnp.float32)
    b2 = b.reshape(1, 1).astype(jnp.float32)
    b_vec = jnp.broadcast_to(b.astype(jnp.float32), (L,))
    za = _tc_matvec(p, w, b2)
    zb = _sc_matvec(p, w_flat, b_vec)
    idx_r = idx.astype(jnp.int32).reshape(NW, N_CHUNK, 128)
    out = _sc_gather(idx_r, za, zb)
    return out.reshape(BATCH, 1)


# speedup vs baseline: 1.6189x; 1.0676x over previous
import functools

import jax
import jax.numpy as jnp
import numpy as np
from jax import lax
from jax.experimental import pallas as pl
from jax.experimental.pallas import tpu as pltpu
from jax.experimental.pallas import tpu_sc as plsc

N_EMB = 1000000
D_EMB = 64
BATCH = 16384

NC = 2
NS = 16
L = 16
NW = NC * NS
B_PER_W = BATCH // NW
N_CHUNK = B_PER_W // 128

BLK = 40960
N_BLK = (N_EMB + BLK - 1) // BLK
Z_LEN = N_BLK * BLK
def _mv_body(p_ref, w_ref, b_ref, z_ref):
    z = lax.dot_general(w_ref[...], p_ref[...], (((1,), (0,)), ((), ())),
                        preferred_element_type=jnp.float32)
    z_ref[...] = z.reshape(BLK) + b_ref[0, 0]


@jax.jit
def _tc_matvec(p, w, b):
    return pl.pallas_call(
        _mv_body,
        grid=(N_BLK,),
        in_specs=[
            pl.BlockSpec((D_EMB, BLK), lambda c: (0, c)),
            pl.BlockSpec((1, D_EMB), lambda c: (0, 0)),
            pl.BlockSpec((1, 1), lambda c: (0, 0)),
        ],
        out_specs=pl.BlockSpec((BLK,), lambda c: (c,)),
        out_shape=jax.ShapeDtypeStruct((Z_LEN,), jnp.float32),
    )(p, w, b)


def _sc_body(idx_hbm, z_hbm, out_hbm, idx_v, out_v, sem):
    cid = lax.axis_index("c")
    sid = lax.axis_index("s")
    wid = sid * NC + cid
    base = wid * B_PER_W

    pltpu.sync_copy(idx_hbm.at[wid], idx_v)
    copies = []
    for j in range(N_CHUNK):
        copies.append(pltpu.async_copy(
            z_hbm.at[idx_v.at[j]],
            out_v.at[pl.ds(j * 128, 128)],
            sem))
    for c in copies:
        c.wait()
    pltpu.sync_copy(out_v, out_hbm.at[pl.ds(base, B_PER_W)])


@jax.jit
def _sc_gather(idx_r, z):
    mesh = plsc.VectorSubcoreMesh(core_axis_name="c", subcore_axis_name="s")
    k = pl.kernel(
        _sc_body,
        mesh=mesh,
        compiler_params=pltpu.CompilerParams(use_tc_tiling_on_sc=False),
        out_type=jax.ShapeDtypeStruct((BATCH,), jnp.float32),
        scratch_types=[
            pltpu.VMEM((N_CHUNK, 128), jnp.int32),
            pltpu.VMEM((B_PER_W,), jnp.float32),
            pltpu.SemaphoreType.DMA,
        ],
    )
    return k(idx_r, z)


def kernel(idx, table, W, b):
    p = table.T
    w = W.reshape(1, D_EMB).astype(jnp.float32)
    b2 = b.reshape(1, 1).astype(jnp.float32)
    z = _tc_matvec(p, w, b2)
    idx_r = idx.astype(jnp.int32).reshape(NW, N_CHUNK, 128)
    out = _sc_gather(idx_r, z)
    return out.reshape(BATCH, 1)
